# baseline (speedup 1.0000x reference)
import jax
import jax.numpy as jnp
from jax import lax
from jax.experimental import pallas as pl
from jax.experimental.pallas import tpu as pltpu
from jax.experimental.pallas import tpu_sc as plsc

NN = 10000
EE = 320000
IN_FT = 128
HID_FT = 256
OUT_FT = 128
EPSV = 1e-5

NC, NS = 2, 16
NPAD = 10240
EPAD = 327680
CH = 64
TRASH = NN + 8
RPS = NPAD // NS

_mesh = plsc.VectorSubcoreMesh(core_axis_name="c", subcore_axis_name="s")


def _fill_zeros_2d(ref, nrows):
    def row(i, _):
        for j in range(128 // 16):
            ref[i, pl.ds(j * 16, 16)] = jnp.zeros((16,), jnp.float32)
        return 0
    lax.fori_loop(0, nrows, row, 0)


def _fill_const_1d(ref, n, val):
    def body(i, _):
        ref[pl.ds(i * 16, 16)] = jnp.full((16,), val, jnp.float32)
        return 0
    lax.fori_loop(0, n // 16, body, 0)


ERows = EPAD // CH


def _deg_body(dst0_hbm, dst1_hbm, dst2_hbm, out0, out1, out2,
              d0, d1, d2, zb, ones_b, idx_b, sem):
    c = lax.axis_index("c")
    s = lax.axis_index("s")
    dsts = [dst0_hbm, dst1_hbm, dst2_hbm]
    outs = [out0, out1, out2]
    degs = [d0, d1, d2]
    _fill_const_1d(zb, RPS, 0.0)
    _fill_const_1d(ones_b, CH, 1.0)
    for v in range(3):
        pltpu.sync_copy(zb, degs[v].at[pl.ds(s * RPS, RPS)])
    plsc.subcore_barrier()
    rpu = ERows // (NC * NS)
    rbase = (c * NS + s) * rpu
    for v in range(3):
        pltpu.sync_copy(dsts[v].at[pl.ds(rbase, rpu)], idx_b)

        def fire(i, _):
            pltpu.async_copy(ones_b, degs[v].at[idx_b.at[i]], sem, add=True)
            return 0
        lax.fori_loop(0, rpu, fire, 0)

        def drain(i, _):
            pltpu.make_async_copy(ones_b, degs[v].at[idx_b.at[0]], sem).wait()
            return 0
        lax.fori_loop(0, rpu, drain, 0)
    plsc.subcore_barrier()
    for v in range(3):
        pltpu.sync_copy(degs[v].at[pl.ds(s * RPS, RPS)],
                        outs[v].at[pl.ds(c * NPAD + s * RPS, RPS)])


_deg_call = pl.kernel(
    _deg_body,
    out_type=(jax.ShapeDtypeStruct((NC * NPAD,), jnp.float32),) * 3,
    mesh=_mesh,
    scratch_types=[
        pltpu.VMEM_SHARED((NPAD,), jnp.float32),
        pltpu.VMEM_SHARED((NPAD,), jnp.float32),
        pltpu.VMEM_SHARED((NPAD,), jnp.float32),
        pltpu.VMEM((RPS,), jnp.float32),
        pltpu.VMEM((CH,), jnp.float32),
        pltpu.VMEM((ERows // (NC * NS), CH), jnp.int32),
        pltpu.SemaphoreType.DMA,
    ],
)


def _make_segsum(feature_split):
    if feature_split:
        nch = ERows // NS
    else:
        nch = ERows // (NC * NS)
    SR = 40
    NB = 4

    def body(*a):
        srcs = [a[0], a[2], a[4]]
        dsts = [a[1], a[3], a[5]]
        tables = [a[6], a[7], a[8]]
        zeros_hbm = a[9]
        outs = [a[10], a[11], a[12]]
        acc_sp, src_b, dst_b = a[13:16]
        rows = list(a[16:20])
        gsem = list(a[20:24])
        ssem = list(a[24:28])
        c = lax.axis_index("c")
        s = lax.axis_index("s")

        if feature_split:
            rbase = s * nch
        else:
            rbase = (c * NS + s) * nch

        for v in range(3):
            src_hbm, dst_hbm = srcs[v], dsts[v]
            table_hbm = tables[v]
            pltpu.async_copy(zeros_hbm.at[pl.ds(s * RPS, RPS)],
                             acc_sp.at[pl.ds(s * RPS, RPS)], gsem[0])
            pltpu.make_async_copy(zeros_hbm.at[pl.ds(s * RPS, RPS)],
                                  acc_sp.at[pl.ds(s * RPS, RPS)],
                                  gsem[0]).wait()
            plsc.subcore_barrier()

            def gath(i, r):
                pltpu.async_copy(table_hbm.at[src_b.at[i]], rows[r], gsem[r])

            def scat_wait(r):
                pltpu.make_async_copy(rows[r], acc_sp.at[dst_b.at[0]],
                                      ssem[r]).wait()

            def stage(t, _):
                tb = rbase + t * SR
                pltpu.sync_copy(src_hbm.at[pl.ds(tb, SR)], src_b)
                pltpu.sync_copy(dst_hbm.at[pl.ds(tb, SR)], dst_b)
                if feature_split:
                    off = c * NN

                    def addoff(i, _):
                        for j in range(CH // 16):
                            src_b[i, pl.ds(j * 16, 16)] = (
                                src_b[i, pl.ds(j * 16, 16)] + off)
                        return 0
                    lax.fori_loop(0, SR, addoff, 0)
                gath(0, 0)
                gath(1, 1)

                def chunkN(i4, _):
                    for r in range(NB):
                        i = i4 * NB + r
                        r2 = (r + 2) % NB

                        @pl.when(i >= 2)
                        def _():
                            scat_wait(r2)

                        @pl.when(i < SR - 2)
                        def _():
                            gath(i + 2, r2)
                        pltpu.make_async_copy(table_hbm.at[src_b.at[i]],
                                              rows[r], gsem[r]).wait()
                        pltpu.async_copy(rows[r], acc_sp.at[dst_b.at[i]],
                                         ssem[r], add=True)
                    return 0
                lax.fori_loop(0, SR // NB, chunkN, 0)
                scat_wait((SR - 2) % NB)
                scat_wait((SR - 1) % NB)
                return 0
            lax.fori_loop(0, nch // SR, stage, 0)
            plsc.subcore_barrier()
            pltpu.sync_copy(acc_sp.at[pl.ds(s * RPS, RPS)],
                            outs[v].at[pl.ds(c * NPAD + s * RPS, RPS)])

    return pl.kernel(
        body,
        out_type=(jax.ShapeDtypeStruct((NC * NPAD, 128), jnp.float32),) * 3,
        mesh=_mesh,
        scratch_types=(
            [pltpu.VMEM_SHARED((NPAD, 128), jnp.float32),
             pltpu.VMEM((40, CH), jnp.int32),
             pltpu.VMEM((40, CH), jnp.int32)]
            + [pltpu.VMEM((CH, 128), jnp.float32)] * 4
            + [pltpu.SemaphoreType.DMA] * 8
        ),
    )


_segsum_l1 = _make_segsum(True)
_segsum_l2 = _make_segsum(False)


def _tc_pre_body(x_ref, w_ref, d0_ref, d1_ref, xs_ref, dinv_ref):
    deg = d0_ref[...] + d1_ref[...] + 1.0
    dinv = lax.rsqrt(deg)
    dinv_ref[...] = dinv
    xw = jnp.dot(x_ref[...], w_ref[...], preferred_element_type=jnp.float32)
    xs = xw * dinv[:NN]
    xs_ref[0] = xs[:, :128]
    xs_ref[1] = xs[:, 128:]


def _tc_pre(x, w1, d0, d1):
    return pl.pallas_call(
        _tc_pre_body,
        out_shape=(
            jax.ShapeDtypeStruct((2, NN, 128), jnp.float32),
            jax.ShapeDtypeStruct((NPAD, 1), jnp.float32),
        ),
    )(x, w1, d0, d1)


def _bn_relu(h, g, be):
    m = jnp.mean(h, axis=0, keepdims=True)
    v = jnp.mean((h - m) * (h - m), axis=0, keepdims=True)
    return jnp.maximum(g * (h - m) * lax.rsqrt(v + EPSV) + be, 0.0)


def _tc_pre3_body(x_ref, w_ref, dg_ref, xs_ref, dinv_ref):
    deg = dg_ref[0, :, 0:1] + dg_ref[0, :, 1:2] + 1.0
    dinv = lax.rsqrt(deg)
    dinv_ref[0] = dinv
    xw = jnp.dot(x_ref[...], w_ref[0], preferred_element_type=jnp.float32)
    xs = xw * dinv[:NN]
    xs_ref[0, 0] = xs[:, :128]
    xs_ref[0, 1] = xs[:, 128:]


_tc_pre3 = pl.pallas_call(
    _tc_pre3_body,
    grid=(3,),
    in_specs=[
        pl.BlockSpec((NN, IN_FT), lambda v: (0, 0)),
        pl.BlockSpec((1, IN_FT, HID_FT), lambda v: (v, 0, 0)),
        pl.BlockSpec((1, NPAD, NC), lambda v: (v, 0, 0)),
    ],
    out_specs=[
        pl.BlockSpec((1, 2, NN, 128), lambda v: (v, 0, 0, 0)),
        pl.BlockSpec((1, NPAD, 1), lambda v: (v, 0, 0)),
    ],
    out_shape=[
        jax.ShapeDtypeStruct((3, 2, NN, 128), jnp.float32),
        jax.ShapeDtypeStruct((3, NPAD, 1), jnp.float32),
    ],
)


def _tc_mid3_body(acc_ref, xs_ref, dinv_ref, b1_ref, g1_ref, be1_ref,
                  w2_ref, xs2_ref):
    dinv = dinv_ref[0][:NN]
    ha = dinv * (acc_ref[0, :NN] + xs_ref[0, 0])
    hb = dinv * (acc_ref[0, NPAD:NPAD + NN] + xs_ref[0, 1])
    h = jnp.concatenate([ha, hb], axis=1) + b1_ref[0]
    h = _bn_relu(h, g1_ref[0], be1_ref[0])
    xs2 = jnp.dot(h, w2_ref[0], preferred_element_type=jnp.float32)
    xs2_ref[0] = xs2 * dinv


_tc_mid3 = pl.pallas_call(
    _tc_mid3_body,
    grid=(3,),
    in_specs=[
        pl.BlockSpec((1, NC * NPAD, 128), lambda v: (v, 0, 0)),
        pl.BlockSpec((1, 2, NN, 128), lambda v: (v, 0, 0, 0)),
        pl.BlockSpec((1, NPAD, 1), lambda v: (v, 0, 0)),
        pl.BlockSpec((1, 1, HID_FT), lambda v: (v, 0, 0)),
        pl.BlockSpec((1, 1, HID_FT), lambda v: (v, 0, 0)),
        pl.BlockSpec((1, 1, HID_FT), lambda v: (v, 0, 0)),
        pl.BlockSpec((1, HID_FT, OUT_FT), lambda v: (v, 0, 0)),
    ],
    out_specs=pl.BlockSpec((1, NN, 128), lambda v: (v, 0, 0)),
    out_shape=jax.ShapeDtypeStruct((3, NN, 128), jnp.float32),
)


def _tc_post3_body(acc_ref, xs2_ref, dinv_ref, b2_ref, g2_ref, be2_ref,
                   mw1_ref, mb1_ref, mw2_ref, mb2_ref, z_ref):
    dinv = dinv_ref[0][:NN]
    pre = dinv * (acc_ref[0, :NN] + acc_ref[0, NPAD:NPAD + NN]
                  + xs2_ref[0]) + b2_ref[0]
    h = _bn_relu(pre, g2_ref[0], be2_ref[0])
    t = jnp.dot(h, mw1_ref[...], preferred_element_type=jnp.float32)
    t = jnp.maximum(t + mb1_ref[...], 0.0)
    z_ref[0] = jnp.dot(t, mw2_ref[...],
                       preferred_element_type=jnp.float32) + mb2_ref[...]


_tc_post3 = pl.pallas_call(
    _tc_post3_body,
    grid=(3,),
    in_specs=[
        pl.BlockSpec((1, NC * NPAD, 128), lambda v: (v, 0, 0)),
        pl.BlockSpec((1, NN, 128), lambda v: (v, 0, 0)),
        pl.BlockSpec((1, NPAD, 1), lambda v: (v, 0, 0)),
        pl.BlockSpec((1, 1, OUT_FT), lambda v: (v, 0, 0)),
        pl.BlockSpec((1, 1, OUT_FT), lambda v: (v, 0, 0)),
        pl.BlockSpec((1, 1, OUT_FT), lambda v: (v, 0, 0)),
        pl.BlockSpec((OUT_FT, OUT_FT), lambda v: (0, 0)),
        pl.BlockSpec((1, OUT_FT), lambda v: (0, 0)),
        pl.BlockSpec((OUT_FT, OUT_FT), lambda v: (0, 0)),
        pl.BlockSpec((1, OUT_FT), lambda v: (0, 0)),
    ],
    out_specs=pl.BlockSpec((1, NN, 128), lambda v: (v, 0, 0)),
    out_shape=jax.ShapeDtypeStruct((3, NN, 128), jnp.float32),
)


def _tc_final3_body(z_ref, cw1_ref, cb1_ref, cw2_ref, cb2_ref, out_ref):
    sm = z_ref[0] + z_ref[1] + z_ref[2]
    t = jnp.dot(sm, cw1_ref[...], preferred_element_type=jnp.float32)
    t = jnp.maximum(t + cb1_ref[...], 0.0)
    out_ref[...] = jnp.dot(t, cw2_ref[...],
                           preferred_element_type=jnp.float32) + cb2_ref[...]


def _tc_final3(z_all, cw1, cb1, cw2, cb2):
    return pl.pallas_call(
        _tc_final3_body,
        out_shape=jax.ShapeDtypeStruct((NN, 10), jnp.float32),
    )(z_all, cw1, cb1, cw2, cb2)


def _tc_mid_body(acc_ref, xs_ref, dinv_ref, b1_ref, g1_ref, be1_ref, w2_ref,
                 xs2_ref):
    dinv = dinv_ref[...][:NN]
    ha = dinv * (acc_ref[0, :NN] + xs_ref[0])
    hb = dinv * (acc_ref[1, :NN] + xs_ref[1])
    h = jnp.concatenate([ha, hb], axis=1) + b1_ref[...]
    h = _bn_relu(h, g1_ref[...], be1_ref[...])
    xs2 = jnp.dot(h, w2_ref[...], preferred_element_type=jnp.float32)
    xs2_ref[...] = xs2 * dinv


def _tc_mid(acc1, xs, dinv, b1, g1, be1, w2):
    return pl.pallas_call(
        _tc_mid_body,
        out_shape=jax.ShapeDtypeStruct((NN, 128), jnp.float32),
    )(acc1, xs, dinv, b1, g1, be1, w2)


def _tc_post_body(acc_ref, xs2_ref, dinv_ref, b2_ref, g2_ref, be2_ref,
                  mw1_ref, mb1_ref, mw2_ref, mb2_ref, z_ref):
    dinv = dinv_ref[...][:NN]
    pre = dinv * (acc_ref[0, :NN] + acc_ref[1, :NN] + xs2_ref[...]) + b2_ref[...]
    h = _bn_relu(pre, g2_ref[...], be2_ref[...])
    t = jnp.dot(h, mw1_ref[...], preferred_element_type=jnp.float32) + mb1_ref[...]
    t = jnp.maximum(t, 0.0)
    z_ref[...] = jnp.dot(t, mw2_ref[...], preferred_element_type=jnp.float32) + mb2_ref[...]


def _tc_post(acc2, xs2, dinv, b2, g2, be2, mw1, mb1, mw2, mb2):
    return pl.pallas_call(
        _tc_post_body,
        out_shape=jax.ShapeDtypeStruct((NN, 128), jnp.float32),
    )(acc2, xs2, dinv, b2, g2, be2, mw1, mb1, mw2, mb2)


def _tc_final_body(z1_ref, z2_ref, z3_ref, cw1_ref, cb1_ref, cw2_ref, cb2_ref,
                   out_ref):
    sm = z1_ref[...] + z2_ref[...] + z3_ref[...]
    t = jnp.dot(sm, cw1_ref[...], preferred_element_type=jnp.float32) + cb1_ref[...]
    t = jnp.maximum(t, 0.0)
    out_ref[...] = jnp.dot(t, cw2_ref[...], preferred_element_type=jnp.float32) + cb2_ref[...]


def _tc_final(z1, z2, z3, cw1, cb1, cw2, cb2):
    return pl.pallas_call(
        _tc_final_body,
        out_shape=jax.ShapeDtypeStruct((NN, 10), jnp.float32),
    )(z1, z2, z3, cw1, cb1, cw2, cb2)


def _pad_edges(ei):
    srcp = jnp.concatenate([ei[0], jnp.zeros((EPAD - EE,), jnp.int32)])
    dstp = jnp.concatenate([ei[1], jnp.full((EPAD - EE,), TRASH, jnp.int32)])
    return srcp.reshape(ERows, CH), dstp.reshape(ERows, CH)


def _views(x, eis, zeros, deg, p):
    deg3 = jnp.stack(deg).reshape(3, NC, NPAD).transpose(0, 2, 1)
    w1s = jnp.stack([p['enc%d_W1' % v] for v in range(3)])
    xs_all, dinv_all = _tc_pre3(x, w1s, deg3)
    a10, a11, a12 = _segsum_l1(
        eis[0][0], eis[0][1], eis[1][0], eis[1][1], eis[2][0], eis[2][1],
        xs_all[0].reshape(2 * NN, 128), xs_all[1].reshape(2 * NN, 128),
        xs_all[2].reshape(2 * NN, 128), zeros)
    acc1_all = jnp.stack([a10, a11, a12])
    xs2_all = jnp.stack([
        _tc_mid(acc1_all[v].reshape(NC, NPAD, 128), xs_all[v], dinv_all[v],
                p['enc%d_b1' % v].reshape(1, -1),
                p['enc%d_g1' % v].reshape(1, -1),
                p['enc%d_be1' % v].reshape(1, -1), p['enc%d_W2' % v])
        for v in range(3)])
    a20, a21, a22 = _segsum_l2(
        eis[0][0], eis[0][1], eis[1][0], eis[1][1], eis[2][0], eis[2][1],
        xs2_all[0], xs2_all[1], xs2_all[2], zeros)
    acc2_all = jnp.stack([a20, a21, a22])
    b2s = jnp.stack([p['enc%d_b2' % v].reshape(1, -1) for v in range(3)])
    g2s = jnp.stack([p['enc%d_g2' % v].reshape(1, -1) for v in range(3)])
    be2s = jnp.stack([p['enc%d_be2' % v].reshape(1, -1) for v in range(3)])
    z_all = _tc_post3(acc2_all, xs2_all, dinv_all, b2s, g2s, be2s,
                      p['mlp_W1'], p['mlp_b1'].reshape(1, -1),
                      p['mlp_W2'], p['mlp_b2'].reshape(1, -1))
    return z_all


def kernel(x, edge_index_local1, edge_index_local2, edge_index_global,
           enc0_W1, enc0_b1, enc0_g1, enc0_be1, enc0_W2, enc0_b2, enc0_g2, enc0_be2,
           enc1_W1, enc1_b1, enc1_g1, enc1_be1, enc1_W2, enc1_b2, enc1_g2, enc1_be2,
           enc2_W1, enc2_b1, enc2_g1, enc2_be1, enc2_W2, enc2_b2, enc2_g2, enc2_be2,
           mlp_W1, mlp_b1, mlp_W2, mlp_b2,
           cls_W1, cls_b1, cls_W2, cls_b2):
    p = {
        'enc0_W1': enc0_W1, 'enc0_b1': enc0_b1, 'enc0_g1': enc0_g1, 'enc0_be1': enc0_be1,
        'enc0_W2': enc0_W2, 'enc0_b2': enc0_b2, 'enc0_g2': enc0_g2, 'enc0_be2': enc0_be2,
        'enc1_W1': enc1_W1, 'enc1_b1': enc1_b1, 'enc1_g1': enc1_g1, 'enc1_be1': enc1_be1,
        'enc1_W2': enc1_W2, 'enc1_b2': enc1_b2, 'enc1_g2': enc1_g2, 'enc1_be2': enc1_be2,
        'enc2_W1': enc2_W1, 'enc2_b1': enc2_b1, 'enc2_g1': enc2_g1, 'enc2_be1': enc2_be1,
        'enc2_W2': enc2_W2, 'enc2_b2': enc2_b2, 'enc2_g2': enc2_g2, 'enc2_be2': enc2_be2,
        'mlp_W1': mlp_W1, 'mlp_b1': mlp_b1, 'mlp_W2': mlp_W2, 'mlp_b2': mlp_b2,
    }
    eis = [_pad_edges(edge_index_local1), _pad_edges(edge_index_local2),
           _pad_edges(edge_index_global)]
    deg = _deg_call(eis[0][1], eis[1][1], eis[2][1])
    zeros = jnp.zeros((NPAD, 128), jnp.float32)
    z_all = _views(x, eis, zeros, deg, p)
    node_pred = _tc_final3(z_all, cls_W1, cls_b1.reshape(1, -1),
                           cls_W2, cls_b2.reshape(1, -1))
    return (z_all[0], z_all[1], z_all[2], node_pred)


# speedup vs baseline: 1.0362x; 1.0362x over previous
import jax
import jax.numpy as jnp
from jax import lax
from jax.experimental import pallas as pl
from jax.experimental.pallas import tpu as pltpu
from jax.experimental.pallas import tpu_sc as plsc

NN = 10000
EE = 320000
IN_FT = 128
HID_FT = 256
OUT_FT = 128
EPSV = 1e-5

NC, NS = 2, 16
NPAD = 10240
EPAD = 327680
CH = 64
TRASH = NN + 8
RPS = NPAD // NS

_mesh = plsc.VectorSubcoreMesh(core_axis_name="c", subcore_axis_name="s")


def _fill_zeros_2d(ref, nrows):
    def row(i, _):
        for j in range(128 // 16):
            ref[i, pl.ds(j * 16, 16)] = jnp.zeros((16,), jnp.float32)
        return 0
    lax.fori_loop(0, nrows, row, 0)


def _fill_const_1d(ref, n, val):
    def body(i, _):
        ref[pl.ds(i * 16, 16)] = jnp.full((16,), val, jnp.float32)
        return 0
    lax.fori_loop(0, n // 16, body, 0)


ERows = EPAD // CH


def _deg_body(dst0_hbm, dst1_hbm, dst2_hbm, out0, out1, out2,
              d0, d1, d2, zb, ones_b, idx_b, sem):
    c = lax.axis_index("c")
    s = lax.axis_index("s")
    dsts = [dst0_hbm, dst1_hbm, dst2_hbm]
    outs = [out0, out1, out2]
    degs = [d0, d1, d2]
    _fill_const_1d(zb, RPS, 0.0)
    _fill_const_1d(ones_b, CH, 1.0)
    for v in range(3):
        pltpu.sync_copy(zb, degs[v].at[pl.ds(s * RPS, RPS)])
    plsc.subcore_barrier()
    rpu = ERows // (NC * NS)
    rbase = (c * NS + s) * rpu
    for v in range(3):
        pltpu.sync_copy(dsts[v].at[pl.ds(rbase, rpu)], idx_b)

        def fire(i, _):
            pltpu.async_copy(ones_b, degs[v].at[idx_b.at[i]], sem, add=True)
            return 0
        lax.fori_loop(0, rpu, fire, 0)

        def drain(i, _):
            pltpu.make_async_copy(ones_b, degs[v].at[idx_b.at[0]], sem).wait()
            return 0
        lax.fori_loop(0, rpu, drain, 0)
    plsc.subcore_barrier()
    for v in range(3):
        pltpu.sync_copy(degs[v].at[pl.ds(s * RPS, RPS)],
                        outs[v].at[pl.ds(c * NPAD + s * RPS, RPS)])


_deg_call = pl.kernel(
    _deg_body,
    out_type=(jax.ShapeDtypeStruct((NC * NPAD,), jnp.float32),) * 3,
    mesh=_mesh,
    scratch_types=[
        pltpu.VMEM_SHARED((NPAD,), jnp.float32),
        pltpu.VMEM_SHARED((NPAD,), jnp.float32),
        pltpu.VMEM_SHARED((NPAD,), jnp.float32),
        pltpu.VMEM((RPS,), jnp.float32),
        pltpu.VMEM((CH,), jnp.float32),
        pltpu.VMEM((ERows // (NC * NS), CH), jnp.int32),
        pltpu.SemaphoreType.DMA,
    ],
)


def _make_segsum(feature_split):
    if feature_split:
        nch = ERows // NS
    else:
        nch = ERows // (NC * NS)
    SR = 40
    NB = 4

    def body(*a):
        srcs = [a[0], a[2], a[4]]
        dsts = [a[1], a[3], a[5]]
        tables = [a[6], a[7], a[8]]
        zeros_hbm = a[9]
        outs = [a[10], a[11], a[12]]
        acc_sp, src_b, dst_b = a[13:16]
        rows = list(a[16:20])
        gsem = list(a[20:24])
        ssem = list(a[24:28])
        c = lax.axis_index("c")
        s = lax.axis_index("s")

        if feature_split:
            rbase = s * nch
        else:
            rbase = (c * NS + s) * nch

        for v in range(3):
            src_hbm, dst_hbm = srcs[v], dsts[v]
            table_hbm = tables[v]
            pltpu.async_copy(zeros_hbm.at[pl.ds(s * RPS, RPS)],
                             acc_sp.at[pl.ds(s * RPS, RPS)], gsem[0])
            pltpu.make_async_copy(zeros_hbm.at[pl.ds(s * RPS, RPS)],
                                  acc_sp.at[pl.ds(s * RPS, RPS)],
                                  gsem[0]).wait()
            plsc.subcore_barrier()

            def gath(i, r):
                pltpu.async_copy(table_hbm.at[src_b.at[i]], rows[r], gsem[r])

            def scat_wait(r):
                pltpu.make_async_copy(rows[r], acc_sp.at[dst_b.at[0]],
                                      ssem[r]).wait()

            def stage(t, _):
                tb = rbase + t * SR
                pltpu.sync_copy(src_hbm.at[pl.ds(tb, SR)], src_b)
                pltpu.sync_copy(dst_hbm.at[pl.ds(tb, SR)], dst_b)
                if feature_split:
                    off = c * NN

                    def addoff(i, _):
                        for j in range(CH // 16):
                            src_b[i, pl.ds(j * 16, 16)] = (
                                src_b[i, pl.ds(j * 16, 16)] + off)
                        return 0
                    lax.fori_loop(0, SR, addoff, 0)
                gath(0, 0)
                gath(1, 1)

                def chunkN(i4, _):
                    for r in range(NB):
                        i = i4 * NB + r
                        r2 = (r + 2) % NB

                        @pl.when(i >= 2)
                        def _():
                            scat_wait(r2)

                        @pl.when(i < SR - 2)
                        def _():
                            gath(i + 2, r2)
                        pltpu.make_async_copy(table_hbm.at[src_b.at[i]],
                                              rows[r], gsem[r]).wait()
                        pltpu.async_copy(rows[r], acc_sp.at[dst_b.at[i]],
                                         ssem[r], add=True)
                    return 0
                lax.fori_loop(0, SR // NB, chunkN, 0)
                scat_wait((SR - 2) % NB)
                scat_wait((SR - 1) % NB)
                return 0
            lax.fori_loop(0, nch // SR, stage, 0)
            plsc.subcore_barrier()
            pltpu.sync_copy(acc_sp.at[pl.ds(s * RPS, RPS)],
                            outs[v].at[pl.ds(c * NPAD + s * RPS, RPS)])

    return pl.kernel(
        body,
        out_type=(jax.ShapeDtypeStruct((NC * NPAD, 128), jnp.float32),) * 3,
        mesh=_mesh,
        scratch_types=(
            [pltpu.VMEM_SHARED((NPAD, 128), jnp.float32),
             pltpu.VMEM((40, CH), jnp.int32),
             pltpu.VMEM((40, CH), jnp.int32)]
            + [pltpu.VMEM((CH, 128), jnp.float32)] * 4
            + [pltpu.SemaphoreType.DMA] * 8
        ),
    )


_segsum_l1 = _make_segsum(True)
_segsum_l2 = _make_segsum(False)


def _tc_pre_body(x_ref, w_ref, d0_ref, d1_ref, xs_ref, dinv_ref):
    deg = d0_ref[...] + d1_ref[...] + 1.0
    dinv = lax.rsqrt(deg)
    dinv_ref[...] = dinv
    xw = jnp.dot(x_ref[...], w_ref[...], preferred_element_type=jnp.float32)
    xs = xw * dinv[:NN]
    xs_ref[0] = xs[:, :128]
    xs_ref[1] = xs[:, 128:]


def _tc_pre(x, w1, d0, d1):
    return pl.pallas_call(
        _tc_pre_body,
        out_shape=(
            jax.ShapeDtypeStruct((2, NN, 128), jnp.float32),
            jax.ShapeDtypeStruct((NPAD, 1), jnp.float32),
        ),
    )(x, w1, d0, d1)


def _bn_relu(h, g, be):
    m = jnp.mean(h, axis=0, keepdims=True)
    v = jnp.mean((h - m) * (h - m), axis=0, keepdims=True)
    return jnp.maximum(g * (h - m) * lax.rsqrt(v + EPSV) + be, 0.0)


def _tc_mid_body(acc_ref, xs_ref, dinv_ref, b1_ref, g1_ref, be1_ref, w2_ref,
                 xs2_ref):
    dinv = dinv_ref[...][:NN]
    ha = dinv * (acc_ref[0, :NN] + xs_ref[0])
    hb = dinv * (acc_ref[1, :NN] + xs_ref[1])
    h = jnp.concatenate([ha, hb], axis=1) + b1_ref[...]
    h = _bn_relu(h, g1_ref[...], be1_ref[...])
    xs2 = jnp.dot(h, w2_ref[...], preferred_element_type=jnp.float32)
    xs2_ref[...] = xs2 * dinv


def _tc_mid(acc1, xs, dinv, b1, g1, be1, w2):
    return pl.pallas_call(
        _tc_mid_body,
        out_shape=jax.ShapeDtypeStruct((NN, 128), jnp.float32),
    )(acc1, xs, dinv, b1, g1, be1, w2)


def _tc_post_body(acc_ref, xs2_ref, dinv_ref, b2_ref, g2_ref, be2_ref,
                  mw1_ref, mb1_ref, mw2_ref, mb2_ref, z_ref):
    dinv = dinv_ref[...][:NN]
    pre = dinv * (acc_ref[0, :NN] + acc_ref[1, :NN] + xs2_ref[...]) + b2_ref[...]
    h = _bn_relu(pre, g2_ref[...], be2_ref[...])
    t = jnp.dot(h, mw1_ref[...], preferred_element_type=jnp.float32) + mb1_ref[...]
    t = jnp.maximum(t, 0.0)
    z_ref[...] = jnp.dot(t, mw2_ref[...], preferred_element_type=jnp.float32) + mb2_ref[...]


def _tc_post(acc2, xs2, dinv, b2, g2, be2, mw1, mb1, mw2, mb2):
    return pl.pallas_call(
        _tc_post_body,
        out_shape=jax.ShapeDtypeStruct((NN, 128), jnp.float32),
    )(acc2, xs2, dinv, b2, g2, be2, mw1, mb1, mw2, mb2)


def _tc_final_body(z1_ref, z2_ref, z3_ref, cw1_ref, cb1_ref, cw2_ref, cb2_ref,
                   out_ref):
    sm = z1_ref[...] + z2_ref[...] + z3_ref[...]
    t = jnp.dot(sm, cw1_ref[...], preferred_element_type=jnp.float32) + cb1_ref[...]
    t = jnp.maximum(t, 0.0)
    out_ref[...] = jnp.dot(t, cw2_ref[...], preferred_element_type=jnp.float32) + cb2_ref[...]


def _tc_final(z1, z2, z3, cw1, cb1, cw2, cb2):
    return pl.pallas_call(
        _tc_final_body,
        out_shape=jax.ShapeDtypeStruct((NN, 10), jnp.float32),
    )(z1, z2, z3, cw1, cb1, cw2, cb2)


def _pad_edges(ei):
    srcp = jnp.concatenate([ei[0], jnp.zeros((EPAD - EE,), jnp.int32)])
    dstp = jnp.concatenate([ei[1], jnp.full((EPAD - EE,), TRASH, jnp.int32)])
    return srcp.reshape(ERows, CH), dstp.reshape(ERows, CH)


def _views(x, eis, zeros, deg, p):
    xs, dinv = {}, {}
    for v in range(3):
        dv = deg[v].reshape(NC, NPAD)
        xs[v], dinv[v] = _tc_pre(x, p['enc%d_W1' % v],
                                 dv[0].reshape(NPAD, 1),
                                 dv[1].reshape(NPAD, 1))
    a10, a11, a12 = _segsum_l1(
        eis[0][0], eis[0][1], eis[1][0], eis[1][1], eis[2][0], eis[2][1],
        xs[0].reshape(2 * NN, 128), xs[1].reshape(2 * NN, 128),
        xs[2].reshape(2 * NN, 128), zeros)
    acc1 = {0: a10, 1: a11, 2: a12}
    xs2 = {}
    for v in range(3):
        pre = 'enc%d' % v
        xs2[v] = _tc_mid(acc1[v].reshape(NC, NPAD, 128), xs[v], dinv[v],
                         p[pre + '_b1'].reshape(1, -1),
                         p[pre + '_g1'].reshape(1, -1),
                         p[pre + '_be1'].reshape(1, -1), p[pre + '_W2'])
    a20, a21, a22 = _segsum_l2(
        eis[0][0], eis[0][1], eis[1][0], eis[1][1], eis[2][0], eis[2][1],
        xs2[0], xs2[1], xs2[2], zeros)
    acc2 = {0: a20, 1: a21, 2: a22}
    zs = []
    for v in range(3):
        pre = 'enc%d' % v
        zs.append(_tc_post(acc2[v].reshape(NC, NPAD, 128), xs2[v], dinv[v],
                           p[pre + '_b2'].reshape(1, -1),
                           p[pre + '_g2'].reshape(1, -1),
                           p[pre + '_be2'].reshape(1, -1),
                           p['mlp_W1'], p['mlp_b1'].reshape(1, -1),
                           p['mlp_W2'], p['mlp_b2'].reshape(1, -1)))
    return zs


def kernel(x, edge_index_local1, edge_index_local2, edge_index_global,
           enc0_W1, enc0_b1, enc0_g1, enc0_be1, enc0_W2, enc0_b2, enc0_g2, enc0_be2,
           enc1_W1, enc1_b1, enc1_g1, enc1_be1, enc1_W2, enc1_b2, enc1_g2, enc1_be2,
           enc2_W1, enc2_b1, enc2_g1, enc2_be1, enc2_W2, enc2_b2, enc2_g2, enc2_be2,
           mlp_W1, mlp_b1, mlp_W2, mlp_b2,
           cls_W1, cls_b1, cls_W2, cls_b2):
    p = {
        'enc0_W1': enc0_W1, 'enc0_b1': enc0_b1, 'enc0_g1': enc0_g1, 'enc0_be1': enc0_be1,
        'enc0_W2': enc0_W2, 'enc0_b2': enc0_b2, 'enc0_g2': enc0_g2, 'enc0_be2': enc0_be2,
        'enc1_W1': enc1_W1, 'enc1_b1': enc1_b1, 'enc1_g1': enc1_g1, 'enc1_be1': enc1_be1,
        'enc1_W2': enc1_W2, 'enc1_b2': enc1_b2, 'enc1_g2': enc1_g2, 'enc1_be2': enc1_be2,
        'enc2_W1': enc2_W1, 'enc2_b1': enc2_b1, 'enc2_g1': enc2_g1, 'enc2_be1': enc2_be1,
        'enc2_W2': enc2_W2, 'enc2_b2': enc2_b2, 'enc2_g2': enc2_g2, 'enc2_be2': enc2_be2,
        'mlp_W1': mlp_W1, 'mlp_b1': mlp_b1, 'mlp_W2': mlp_W2, 'mlp_b2': mlp_b2,
    }
    eis = [_pad_edges(edge_index_local1), _pad_edges(edge_index_local2),
           _pad_edges(edge_index_global)]
    deg = _deg_call(eis[0][1], eis[1][1], eis[2][1])
    zeros = jnp.zeros((NPAD, 128), jnp.float32)
    zs = _views(x, eis, zeros, deg, p)
    node_pred = _tc_final(zs[0], zs[1], zs[2],
                          cls_W1, cls_b1.reshape(1, -1),
                          cls_W2, cls_b2.reshape(1, -1))
    return (zs[0], zs[1], zs[2], node_pred)
